# Initial kernel scaffold; baseline (speedup 1.0000x reference)
import jax
import jax.numpy as jnp
from jax.experimental import pallas as pl


def kernel(tokens, table):
    raise NotImplementedError("write your pallas kernel here")


# speedup vs baseline: 37.5220x; 37.5220x over previous
import functools

import jax
import jax.numpy as jnp
from jax import lax
from jax.experimental import pallas as pl
from jax.experimental.pallas import tpu as pltpu
from jax.experimental.pallas import tpu_sc as plsc

_VOCAB = 1001
_OOV = 100
_TBL = _VOCAB + _OOV
_N = 4096 * 50
_NC = 2
_NS = 16
_NW = _NC * _NS
_CHUNK = _N // _NW
_L = 16


def _sc_body(tok_hbm, tbl_hbm, out_hbm, tok_v, out_v, tbl_v):
    wid = lax.axis_index("s") * _NC + lax.axis_index("c")
    base = wid * _CHUNK
    pltpu.sync_copy(tbl_hbm, tbl_v)
    pltpu.sync_copy(tok_hbm.at[pl.ds(base, _CHUNK)], tok_v)

    def step(i, carry):
        tok = tok_v[pl.ds(i * _L, _L)]
        idx = jnp.where(tok < _VOCAB, tok, _VOCAB + lax.rem(tok, _OOV))
        out_v[pl.ds(i * _L, _L)] = plsc.load_gather(tbl_v, [idx])
        return carry

    lax.fori_loop(0, _CHUNK // _L, step, 0)
    pltpu.sync_copy(out_v, out_hbm.at[pl.ds(base, _CHUNK)])


@jax.jit
def kernel(tokens, table):
    mesh = plsc.VectorSubcoreMesh(core_axis_name="c", subcore_axis_name="s")
    out = pl.kernel(
        _sc_body,
        out_type=jax.ShapeDtypeStruct((_N,), jnp.float32),
        mesh=mesh,
        compiler_params=pltpu.CompilerParams(needs_layout_passes=False),
        scratch_types=[
            pltpu.VMEM((_CHUNK,), jnp.int32),
            pltpu.VMEM((_CHUNK,), jnp.float32),
            pltpu.VMEM((_TBL,), jnp.float32),
        ],
    )(tokens.reshape(-1), table)
    return out.reshape(tokens.shape)
